# baseline (speedup 1.0000x reference)
import functools

import jax
import jax.numpy as jnp
from jax import lax
from jax.experimental import pallas as pl
from jax.experimental.pallas import tpu as pltpu
from jax.experimental.pallas import tpu_sc as plsc

N_NODES = 10000
N_EDGES = 320000
D_IN = 128
D_HID = 128
N_CLASSES = 64

NC, NS, L = 2, 16, 16
NW = NC * NS
N_PAD = 10240
ROWS_PER_TILE = N_PAD // NS
CH = 128
NCHUNK = 80
EPW = NCHUNK * CH
E_PAD = NW * EPW

_MESH = plsc.VectorSubcoreMesh(core_axis_name="c", subcore_axis_name="s")


@functools.partial(
    pl.kernel,
    mesh=_MESH,
    compiler_params=pltpu.CompilerParams(use_tc_tiling_on_sc=False),
    out_type=jax.ShapeDtypeStruct((NC * N_PAD,), jnp.float32),
    scratch_types=[
        pltpu.VMEM((NCHUNK, CH), jnp.int32),
        pltpu.VMEM((NCHUNK, CH), jnp.float32),
        pltpu.VMEM((ROWS_PER_TILE,), jnp.float32),
        pltpu.VMEM_SHARED((N_PAD,), jnp.float32),
        pltpu.SemaphoreType.DMA,
    ],
)
def _deg_kernel(dst_hbm, ew_hbm, out_hbm, didx, ewv, zbuf, acc, sem):
    cid = lax.axis_index("c")
    sid = lax.axis_index("s")
    wid = cid * NS + sid

    def _zero(i, _):
        zbuf[pl.ds(i * L, L)] = jnp.zeros((L,), jnp.float32)
        return 0

    lax.fori_loop(0, ROWS_PER_TILE // L, _zero, 0)
    pltpu.sync_copy(zbuf, acc.at[pl.ds(sid * ROWS_PER_TILE, ROWS_PER_TILE)])
    pltpu.sync_copy(dst_hbm.at[wid], didx)
    pltpu.sync_copy(ew_hbm.at[wid], ewv)
    plsc.subcore_barrier()

    def _fire(k, _):
        pltpu.async_copy(ewv.at[k], acc.at[didx.at[k]], sem, add=True)
        return 0

    lax.fori_loop(0, NCHUNK, _fire, 0)

    def _drain(k, _):
        pltpu.make_async_copy(ewv.at[0], acc.at[didx.at[0]], sem).wait()
        return 0

    lax.fori_loop(0, NCHUNK, _drain, 0)
    plsc.subcore_barrier()
    pltpu.sync_copy(
        acc.at[pl.ds(sid * ROWS_PER_TILE, ROWS_PER_TILE)],
        out_hbm.at[pl.ds(cid * N_PAD + sid * ROWS_PER_TILE, ROWS_PER_TILE)],
    )


def _make_agg(D, NB, CHK, NCHK):
    NQ = NCHK // NB

    @functools.partial(
        pl.kernel,
        mesh=_MESH,
        compiler_params=pltpu.CompilerParams(use_tc_tiling_on_sc=False),
        out_type=jax.ShapeDtypeStruct((NC * N_PAD, D), jnp.float32),
        scratch_types=[
            pltpu.VMEM((NB, CHK), jnp.int32),
            pltpu.VMEM((NCHK, CHK), jnp.int32),
            pltpu.VMEM((NB, CHK), jnp.float32),
            pltpu.VMEM((NB, CHK, D), jnp.float32),
            pltpu.VMEM((8, D), jnp.float32),
            pltpu.VMEM_SHARED((N_PAD, D), jnp.float32),
        ] + [pltpu.SemaphoreType.DMA] * (3 * NB),
    )
    def _agg(h_hbm, src_hbm, dst_hbm, ew_hbm, out_hbm,
             sidx, didx, ewr, msg, zbuf, acc, *sems):
        sem_i = sems[0:NB]
        sem_g = sems[NB:2 * NB]
        sem_s = sems[2 * NB:3 * NB]
        cid = lax.axis_index("c")
        sid = lax.axis_index("s")
        wid = cid * NS + sid

        for i in range(8):
            for j in range(D // L):
                zbuf[i, pl.ds(j * L, L)] = jnp.zeros((L,), jnp.float32)

        def _zacc(i, _):
            pltpu.sync_copy(
                zbuf, acc.at[pl.ds(sid * ROWS_PER_TILE + i * 8, 8)])
            return 0

        lax.fori_loop(0, ROWS_PER_TILE // 8, _zacc, 0)
        pltpu.sync_copy(dst_hbm.at[wid], didx)

        def _fire_idx(k, b):
            pltpu.async_copy(src_hbm.at[wid, k], sidx.at[b], sem_i[b])
            pltpu.async_copy(ew_hbm.at[wid, k], ewr.at[b], sem_i[b])

        def _wait_idx(k, b):
            pltpu.make_async_copy(src_hbm.at[wid, k], sidx.at[b],
                                  sem_i[b]).wait()
            pltpu.make_async_copy(ew_hbm.at[wid, k], ewr.at[b],
                                  sem_i[b]).wait()

        def _fire_gather(b):
            pltpu.async_copy(h_hbm.at[sidx.at[b]], msg.at[b], sem_g[b])

        def _wait_gather(b):
            pltpu.make_async_copy(h_hbm.at[sidx.at[b]], msg.at[b],
                                  sem_g[b]).wait()

        def _fire_scatter(k, b):
            pltpu.async_copy(msg.at[b], acc.at[didx.at[k]], sem_s[b],
                             add=True)

        def _wait_scatter(k, b):
            pltpu.make_async_copy(msg.at[b], acc.at[didx.at[k]],
                                  sem_s[b]).wait()

        plsc.subcore_barrier()

        for b in range(NB):
            _fire_idx(b, b)
        _wait_idx(0, 0)
        _fire_gather(0)

        def _round(q, _):
            for i in range(NB):
                k = q * NB + i
                b = i
                b1 = (i + 1) % NB
                _wait_gather(b)

                @pl.when(k < NCHK - 1)
                def _(k=k, b1=b1):
                    _wait_idx(k + 1, b1)

                    @pl.when(k + 1 - NB >= 0)
                    def _():
                        _wait_scatter(k + 1 - NB, b1)

                    _fire_gather(b1)

                def _group(g, _, b=b):
                    w = ewr[b, pl.ds(g * L, L)]
                    for r in range(L):
                        s = w[r]
                        e = g * L + r
                        for j in range(D // L):
                            sl = pl.ds(j * L, L)
                            msg[b, e, sl] = msg[b, e, sl] * s
                    return 0

                lax.fori_loop(0, CHK // L, _group, 0)
                _fire_scatter(k, b)

                @pl.when(k + NB < NCHK)
                def _(k=k, b=b):
                    _fire_idx(k + NB, b)

            return 0

        lax.fori_loop(0, NQ, _round, 0)
        for i in range(NB):
            _wait_scatter(NCHK - NB + i, i)
        plsc.subcore_barrier()
        pltpu.sync_copy(
            acc.at[pl.ds(sid * ROWS_PER_TILE, ROWS_PER_TILE)],
            out_hbm.at[pl.ds(cid * N_PAD + sid * ROWS_PER_TILE, ROWS_PER_TILE)],
        )

    return _agg


_agg128 = _make_agg(D_HID, 4, 64, 2 * NCHUNK)
_agg64 = _make_agg(N_CLASSES, 8, CH, NCHUNK)


_BLK = 512
_NBLK = N_PAD // _BLK

_P = lax.Precision.HIGHEST


def _pre1_body(deg0_ref, deg1_ref, x_ref, w1_ref, dis_ref, h1p_ref):
    deg = deg0_ref[...] + deg1_ref[...]
    dis = jnp.where(deg > 0, lax.rsqrt(jnp.maximum(deg, 1e-12)), 0.0)
    dis_ref[...] = dis
    h = jnp.dot(x_ref[...], w1_ref[...], preferred_element_type=jnp.float32,
                precision=_P)
    h1p_ref[...] = h * dis[:, None]


def _pre1(deg0, deg1, x, w1):
    return pl.pallas_call(
        _pre1_body,
        grid=(_NBLK,),
        in_specs=[
            pl.BlockSpec((_BLK,), lambda i: (i,)),
            pl.BlockSpec((_BLK,), lambda i: (i,)),
            pl.BlockSpec((_BLK, D_IN), lambda i: (i, 0)),
            pl.BlockSpec((D_IN, D_HID), lambda i: (0, 0)),
        ],
        out_specs=[
            pl.BlockSpec((_BLK,), lambda i: (i,)),
            pl.BlockSpec((_BLK, D_HID), lambda i: (i, 0)),
        ],
        out_shape=[
            jax.ShapeDtypeStruct((N_PAD,), jnp.float32),
            jax.ShapeDtypeStruct((N_PAD, D_HID), jnp.float32),
        ],
    )(deg0, deg1, x, w1)


def _mid_body(a0_ref, a1_ref, dis_ref, b1_ref, w2_ref, out1_ref, h2p_ref):
    dis = dis_ref[...]
    t = (a0_ref[...] + a1_ref[...]) * dis[:, None] + b1_ref[...]
    out1 = jnp.maximum(t, 0.0)
    out1_ref[...] = out1
    h2 = jnp.dot(out1, w2_ref[...], preferred_element_type=jnp.float32,
                 precision=_P)
    h2p_ref[...] = h2 * dis[:, None]


def _mid(a0, a1, dis, b1, w2):
    return pl.pallas_call(
        _mid_body,
        grid=(_NBLK,),
        in_specs=[
            pl.BlockSpec((_BLK, D_HID), lambda i: (i, 0)),
            pl.BlockSpec((_BLK, D_HID), lambda i: (i, 0)),
            pl.BlockSpec((_BLK,), lambda i: (i,)),
            pl.BlockSpec((1, D_HID), lambda i: (0, 0)),
            pl.BlockSpec((D_HID, N_CLASSES), lambda i: (0, 0)),
        ],
        out_specs=[
            pl.BlockSpec((_BLK, D_HID), lambda i: (i, 0)),
            pl.BlockSpec((_BLK, N_CLASSES), lambda i: (i, 0)),
        ],
        out_shape=[
            jax.ShapeDtypeStruct((N_PAD, D_HID), jnp.float32),
            jax.ShapeDtypeStruct((N_PAD, N_CLASSES), jnp.float32),
        ],
    )(a0, a1, dis, b1, w2)


def _post_body(a0_ref, a1_ref, dis_ref, b2_ref, out2_ref):
    dis = dis_ref[...]
    out2_ref[...] = (a0_ref[...] + a1_ref[...]) * dis[:, None] + b2_ref[...]


def _post(a0, a1, dis, b2):
    return pl.pallas_call(
        _post_body,
        grid=(_NBLK,),
        in_specs=[
            pl.BlockSpec((_BLK, N_CLASSES), lambda i: (i, 0)),
            pl.BlockSpec((_BLK, N_CLASSES), lambda i: (i, 0)),
            pl.BlockSpec((_BLK,), lambda i: (i,)),
            pl.BlockSpec((1, N_CLASSES), lambda i: (0, 0)),
        ],
        out_specs=pl.BlockSpec((_BLK, N_CLASSES), lambda i: (i, 0)),
        out_shape=jax.ShapeDtypeStruct((N_PAD, N_CLASSES), jnp.float32),
    )(a0, a1, dis, b2)


def kernel(x, edge_index, edge_weight, W1, b1, W2, b2):
    src = edge_index[0].astype(jnp.int32)
    dst = edge_index[1].astype(jnp.int32)
    ew = edge_weight.astype(jnp.float32)
    epad = E_PAD - N_EDGES
    src = jnp.concatenate(
        [src, jnp.zeros((epad,), jnp.int32)]).reshape(NW, NCHUNK, CH)
    dst = jnp.concatenate(
        [dst, jnp.full((epad,), N_NODES, jnp.int32)]).reshape(NW, NCHUNK, CH)
    ew = jnp.concatenate(
        [ew, jnp.zeros((epad,), jnp.float32)]).reshape(NW, NCHUNK, CH)
    x_pad = jnp.pad(x, ((0, N_PAD - N_NODES), (0, 0)))

    src2 = src.reshape(NW, 2 * NCHUNK, CH // 2)
    dst2 = dst.reshape(NW, 2 * NCHUNK, CH // 2)
    ew2 = ew.reshape(NW, 2 * NCHUNK, CH // 2)
    degp = _deg_kernel(dst, ew)
    dis, h1p = _pre1(degp[:N_PAD], degp[N_PAD:], x_pad, W1)
    agg1 = _agg128(h1p, src2, dst2, ew2)
    out1, h2p = _mid(agg1[:N_PAD], agg1[N_PAD:], dis,
                     b1.reshape(1, D_HID), W2)
    agg2 = _agg64(h2p, src, dst, ew)
    out2 = _post(agg2[:N_PAD], agg2[N_PAD:], dis, b2.reshape(1, N_CLASSES))
    return (x, out1[:N_NODES], out2[:N_NODES])


# speedup vs baseline: 1.3298x; 1.3298x over previous
import functools

import jax
import jax.numpy as jnp
from jax import lax
from jax.experimental import pallas as pl
from jax.experimental.pallas import tpu as pltpu
from jax.experimental.pallas import tpu_sc as plsc

N_NODES = 10000
N_EDGES = 320000
D_IN = 128
D_HID = 128
N_CLASSES = 64

NC, NS, L = 2, 16, 16
NW = NC * NS
N_PAD = 10240
ROWS_PER_TILE = N_PAD // NS
CH = 128
NCHUNK = 80
E_PAD = NW * NCHUNK * CH
FAST_CID = 0
KF128, KS128 = 112, 48
KF64, KS64 = 236, 84
E_TOT = (E_PAD // CH + KF128 + 8) * CH

_MESH = plsc.VectorSubcoreMesh(core_axis_name="c", subcore_axis_name="s")


def _split(cid, sid, kf, ks):
    kbase = jnp.where(cid == FAST_CID, sid * kf, NS * kf + sid * ks)
    kcnt = jnp.where(cid == FAST_CID, kf, ks)
    return kbase, kcnt


@functools.partial(
    pl.kernel,
    mesh=_MESH,
    compiler_params=pltpu.CompilerParams(use_tc_tiling_on_sc=False),
    out_type=[jax.ShapeDtypeStruct((N_PAD,), jnp.float32),
              jax.ShapeDtypeStruct((N_PAD,), jnp.float32)],
    scratch_types=[
        pltpu.VMEM((KF128, CH), jnp.int32),
        pltpu.VMEM((KF128, CH), jnp.float32),
        pltpu.VMEM((ROWS_PER_TILE,), jnp.float32),
        pltpu.VMEM_SHARED((N_PAD,), jnp.float32),
        pltpu.SemaphoreType.DMA,
    ],
)
def _deg_kernel(dst_hbm, ew_hbm, out0_hbm, out1_hbm, didx, ewv, zbuf, acc,
                sem):
    cid = lax.axis_index("c")
    sid = lax.axis_index("s")
    kbase, kcnt = _split(cid, sid, KF128, KS128)

    def _zero(i, _):
        zbuf[pl.ds(i * L, L)] = jnp.zeros((L,), jnp.float32)
        return 0

    lax.fori_loop(0, ROWS_PER_TILE // L, _zero, 0)
    pltpu.sync_copy(zbuf, acc.at[pl.ds(sid * ROWS_PER_TILE, ROWS_PER_TILE)])
    pltpu.sync_copy(dst_hbm.at[pl.ds(kbase, KF128)], didx)
    pltpu.sync_copy(ew_hbm.at[pl.ds(kbase, KF128)], ewv)
    plsc.subcore_barrier()

    def _fire(k, _):
        pltpu.async_copy(ewv.at[k], acc.at[didx.at[k]], sem, add=True)
        return 0

    lax.fori_loop(0, kcnt, _fire, 0)

    def _drain(k, _):
        pltpu.make_async_copy(ewv.at[0], acc.at[didx.at[0]], sem).wait()
        return 0

    lax.fori_loop(0, kcnt, _drain, 0)
    plsc.subcore_barrier()

    @pl.when(cid == 0)
    def _():
        pltpu.sync_copy(
            acc.at[pl.ds(sid * ROWS_PER_TILE, ROWS_PER_TILE)],
            out0_hbm.at[pl.ds(sid * ROWS_PER_TILE, ROWS_PER_TILE)])

    @pl.when(cid == 1)
    def _():
        pltpu.sync_copy(
            acc.at[pl.ds(sid * ROWS_PER_TILE, ROWS_PER_TILE)],
            out1_hbm.at[pl.ds(sid * ROWS_PER_TILE, ROWS_PER_TILE)])


def _make_agg(D, NB, CHK, KF, KS, ZR):

    @functools.partial(
        pl.kernel,
        mesh=_MESH,
        compiler_params=pltpu.CompilerParams(use_tc_tiling_on_sc=False),
        out_type=[jax.ShapeDtypeStruct((N_PAD, D), jnp.float32),
                  jax.ShapeDtypeStruct((N_PAD, D), jnp.float32)],
        scratch_types=[
            pltpu.VMEM((NB, CHK), jnp.int32),
            pltpu.VMEM((KF, CHK), jnp.int32),
            pltpu.VMEM((NB, CHK), jnp.float32),
            pltpu.VMEM((NB, CHK, D), jnp.float32),
            pltpu.VMEM((ZR, D), jnp.float32),
            pltpu.VMEM_SHARED((N_PAD, D), jnp.float32),
        ] + [pltpu.SemaphoreType.DMA] * (3 * NB),
    )
    def _agg(h_hbm, src_hbm, dst_hbm, ew_hbm, out0_hbm, out1_hbm,
             sidx, didx, ewr, msg, zbuf, acc, *sems):
        sem_i = sems[0:NB]
        sem_g = sems[NB:2 * NB]
        sem_s = sems[2 * NB:3 * NB]
        cid = lax.axis_index("c")
        sid = lax.axis_index("s")
        kbase, kcnt = _split(cid, sid, KF, KS)

        for i in range(ZR):
            for j in range(D // L):
                zbuf[i, pl.ds(j * L, L)] = jnp.zeros((L,), jnp.float32)

        def _zacc(i, _):
            pltpu.sync_copy(
                zbuf, acc.at[pl.ds(sid * ROWS_PER_TILE + i * ZR, ZR)])
            return 0

        lax.fori_loop(0, ROWS_PER_TILE // ZR, _zacc, 0)
        pltpu.sync_copy(dst_hbm.at[pl.ds(kbase, KF)], didx)

        def _fire_idx(k, b):
            pltpu.async_copy(src_hbm.at[kbase + k], sidx.at[b], sem_i[b])
            pltpu.async_copy(ew_hbm.at[kbase + k], ewr.at[b], sem_i[b])

        def _wait_idx(k, b):
            pltpu.make_async_copy(src_hbm.at[kbase + k], sidx.at[b],
                                  sem_i[b]).wait()
            pltpu.make_async_copy(ew_hbm.at[kbase + k], ewr.at[b],
                                  sem_i[b]).wait()

        def _fire_gather(b):
            pltpu.async_copy(h_hbm.at[sidx.at[b]], msg.at[b], sem_g[b])

        def _wait_gather(b):
            pltpu.make_async_copy(h_hbm.at[sidx.at[b]], msg.at[b],
                                  sem_g[b]).wait()

        def _fire_scatter(k, b):
            pltpu.async_copy(msg.at[b], acc.at[didx.at[k]], sem_s[b],
                             add=True)

        def _wait_scatter(k, b):
            pltpu.make_async_copy(msg.at[b], acc.at[didx.at[k]],
                                  sem_s[b]).wait()

        plsc.subcore_barrier()

        for b in range(NB):
            _fire_idx(b, b)
        _wait_idx(0, 0)
        _fire_gather(0)

        def _round(q, _):
            for i in range(NB):
                k = q * NB + i
                b = i
                b1 = (i + 1) % NB
                _wait_gather(b)

                @pl.when(k < kcnt - 1)
                def _(k=k, b1=b1):
                    _wait_idx(k + 1, b1)

                    @pl.when(k + 1 - NB >= 0)
                    def _():
                        _wait_scatter(k + 1 - NB, b1)

                    _fire_gather(b1)

                def _group(g, _, b=b):
                    w = ewr[b, pl.ds(g * L, L)]
                    for r in range(L):
                        s = w[r]
                        e = g * L + r
                        for j in range(D // L):
                            sl = pl.ds(j * L, L)
                            msg[b, e, sl] = msg[b, e, sl] * s
                    return 0

                lax.fori_loop(0, CHK // L, _group, 0)
                _fire_scatter(k, b)

                @pl.when(k + NB < kcnt)
                def _(k=k, b=b):
                    _fire_idx(k + NB, b)

            return 0

        lax.fori_loop(0, kcnt // NB, _round, 0)
        for i in range(NB):
            _wait_scatter(kcnt - NB + i, i)
        plsc.subcore_barrier()

        @pl.when(cid == 0)
        def _():
            pltpu.sync_copy(
                acc.at[pl.ds(sid * ROWS_PER_TILE, ROWS_PER_TILE)],
                out0_hbm.at[pl.ds(sid * ROWS_PER_TILE, ROWS_PER_TILE)])

        @pl.when(cid == 1)
        def _():
            pltpu.sync_copy(
                acc.at[pl.ds(sid * ROWS_PER_TILE, ROWS_PER_TILE)],
                out1_hbm.at[pl.ds(sid * ROWS_PER_TILE, ROWS_PER_TILE)])

    return _agg


_agg128 = _make_agg(D_HID, 4, 64, KF64, KS64, 4)
_agg64 = _make_agg(N_CLASSES, 8, CH, KF128, KS128, 8)


_BLK = 512
_NBLK = N_PAD // _BLK

_P = lax.Precision.HIGHEST


def _pre1_body(deg0_ref, deg1_ref, x_ref, w1_ref, dis_ref, h1p_ref):
    deg = deg0_ref[...] + deg1_ref[...]
    dis = jnp.where(deg > 0, lax.rsqrt(jnp.maximum(deg, 1e-12)), 0.0)
    dis_ref[...] = dis
    h = jnp.dot(x_ref[...], w1_ref[...], preferred_element_type=jnp.float32,
                precision=_P)
    h1p_ref[...] = h * dis[:, None]


def _pre1(deg0, deg1, x, w1):
    return pl.pallas_call(
        _pre1_body,
        grid=(_NBLK,),
        in_specs=[
            pl.BlockSpec((_BLK,), lambda i: (i,)),
            pl.BlockSpec((_BLK,), lambda i: (i,)),
            pl.BlockSpec((_BLK, D_IN), lambda i: (i, 0)),
            pl.BlockSpec((D_IN, D_HID), lambda i: (0, 0)),
        ],
        out_specs=[
            pl.BlockSpec((_BLK,), lambda i: (i,)),
            pl.BlockSpec((_BLK, D_HID), lambda i: (i, 0)),
        ],
        out_shape=[
            jax.ShapeDtypeStruct((N_PAD,), jnp.float32),
            jax.ShapeDtypeStruct((N_PAD, D_HID), jnp.float32),
        ],
    )(deg0, deg1, x, w1)


def _mid_body(a0_ref, a1_ref, dis_ref, b1_ref, w2_ref, out1_ref, h2p_ref):
    dis = dis_ref[...]
    t = (a0_ref[...] + a1_ref[...]) * dis[:, None] + b1_ref[...]
    out1 = jnp.maximum(t, 0.0)
    out1_ref[...] = out1
    h2 = jnp.dot(out1, w2_ref[...], preferred_element_type=jnp.float32,
                 precision=_P)
    h2p_ref[...] = h2 * dis[:, None]


def _mid(a0, a1, dis, b1, w2):
    return pl.pallas_call(
        _mid_body,
        grid=(_NBLK,),
        in_specs=[
            pl.BlockSpec((_BLK, D_HID), lambda i: (i, 0)),
            pl.BlockSpec((_BLK, D_HID), lambda i: (i, 0)),
            pl.BlockSpec((_BLK,), lambda i: (i,)),
            pl.BlockSpec((1, D_HID), lambda i: (0, 0)),
            pl.BlockSpec((D_HID, N_CLASSES), lambda i: (0, 0)),
        ],
        out_specs=[
            pl.BlockSpec((_BLK, D_HID), lambda i: (i, 0)),
            pl.BlockSpec((_BLK, N_CLASSES), lambda i: (i, 0)),
        ],
        out_shape=[
            jax.ShapeDtypeStruct((N_PAD, D_HID), jnp.float32),
            jax.ShapeDtypeStruct((N_PAD, N_CLASSES), jnp.float32),
        ],
    )(a0, a1, dis, b1, w2)


def _post_body(a0_ref, a1_ref, dis_ref, b2_ref, out2_ref):
    dis = dis_ref[...]
    out2_ref[...] = (a0_ref[...] + a1_ref[...]) * dis[:, None] + b2_ref[...]


def _post(a0, a1, dis, b2):
    return pl.pallas_call(
        _post_body,
        grid=(_NBLK,),
        in_specs=[
            pl.BlockSpec((_BLK, N_CLASSES), lambda i: (i, 0)),
            pl.BlockSpec((_BLK, N_CLASSES), lambda i: (i, 0)),
            pl.BlockSpec((_BLK,), lambda i: (i,)),
            pl.BlockSpec((1, N_CLASSES), lambda i: (0, 0)),
        ],
        out_specs=pl.BlockSpec((_BLK, N_CLASSES), lambda i: (i, 0)),
        out_shape=jax.ShapeDtypeStruct((N_PAD, N_CLASSES), jnp.float32),
    )(a0, a1, dis, b2)


def kernel(x, edge_index, edge_weight, W1, b1, W2, b2):
    src = edge_index[0].astype(jnp.int32)
    dst = edge_index[1].astype(jnp.int32)
    ew = edge_weight.astype(jnp.float32)
    epad = E_PAD - N_EDGES
    tpad = E_TOT - E_PAD
    src = jnp.concatenate([src, jnp.zeros((epad + tpad,), jnp.int32)])
    dump = N_NODES + jnp.mod(jnp.arange(epad, dtype=jnp.int32),
                             N_PAD - N_NODES)
    dst = jnp.concatenate([dst, dump, jnp.zeros((tpad,), jnp.int32)])
    ew = jnp.concatenate([ew, jnp.zeros((epad + tpad,), jnp.float32)])
    src128 = src.reshape(-1, CH)
    dst128 = dst.reshape(-1, CH)
    ew128 = ew.reshape(-1, CH)
    src64 = src.reshape(-1, CH // 2)
    dst64 = dst.reshape(-1, CH // 2)
    ew64 = ew.reshape(-1, CH // 2)
    x_pad = jnp.pad(x, ((0, N_PAD - N_NODES), (0, 0)))

    deg0, deg1 = _deg_kernel(dst128, ew128)
    dis, h1p = _pre1(deg0, deg1, x_pad, W1)
    a0, a1 = _agg128(h1p, src64, dst64, ew64)
    out1, h2p = _mid(a0, a1, dis, b1.reshape(1, D_HID), W2)
    q0, q1 = _agg64(h2p, src128, dst128, ew128)
    out2 = _post(q0, q1, dis, b2.reshape(1, N_CLASSES))
    return (x, out1[:N_NODES], out2[:N_NODES])


# speedup vs baseline: 1.3404x; 1.0080x over previous
import functools

import jax
import jax.numpy as jnp
from jax import lax
from jax.experimental import pallas as pl
from jax.experimental.pallas import tpu as pltpu
from jax.experimental.pallas import tpu_sc as plsc

N_NODES = 10000
N_EDGES = 320000
D_IN = 128
D_HID = 128
N_CLASSES = 64

NC, NS, L = 2, 16, 16
NW = NC * NS
N_PAD = 10240
ROWS_PER_TILE = N_PAD // NS
CH = 128
NCHUNK = 80
E_PAD = NW * NCHUNK * CH
FAST_CID = 0
KF128, KS128 = 112, 48
KF64, KS64 = 236, 84
E_TOT = (E_PAD // CH + KF128 + 8) * CH

_MESH = plsc.VectorSubcoreMesh(core_axis_name="c", subcore_axis_name="s")


def _split(cid, sid, kf, ks):
    kbase = jnp.where(cid == FAST_CID, sid * kf, NS * kf + sid * ks)
    kcnt = jnp.where(cid == FAST_CID, kf, ks)
    return kbase, kcnt


@functools.partial(
    pl.kernel,
    mesh=_MESH,
    compiler_params=pltpu.CompilerParams(use_tc_tiling_on_sc=False),
    out_type=[jax.ShapeDtypeStruct((N_PAD,), jnp.float32),
              jax.ShapeDtypeStruct((N_PAD,), jnp.float32)],
    scratch_types=[
        pltpu.VMEM((KF128, CH), jnp.int32),
        pltpu.VMEM((KF128, CH), jnp.float32),
        pltpu.VMEM((ROWS_PER_TILE,), jnp.float32),
        pltpu.VMEM_SHARED((N_PAD,), jnp.float32),
        pltpu.SemaphoreType.DMA,
    ],
)
def _deg_kernel(dst_hbm, ew_hbm, out0_hbm, out1_hbm, didx, ewv, zbuf, acc,
                sem):
    cid = lax.axis_index("c")
    sid = lax.axis_index("s")
    kbase, kcnt = _split(cid, sid, KF128, KS128)

    def _zero(i, _):
        zbuf[pl.ds(i * L, L)] = jnp.zeros((L,), jnp.float32)
        return 0

    lax.fori_loop(0, ROWS_PER_TILE // L, _zero, 0)
    pltpu.sync_copy(zbuf, acc.at[pl.ds(sid * ROWS_PER_TILE, ROWS_PER_TILE)])
    pltpu.sync_copy(dst_hbm.at[pl.ds(kbase, KF128)], didx)
    pltpu.sync_copy(ew_hbm.at[pl.ds(kbase, KF128)], ewv)
    plsc.subcore_barrier()

    def _fire(k, _):
        pltpu.async_copy(ewv.at[k], acc.at[didx.at[k]], sem, add=True)
        return 0

    lax.fori_loop(0, kcnt, _fire, 0)

    def _drain(k, _):
        pltpu.make_async_copy(ewv.at[0], acc.at[didx.at[0]], sem).wait()
        return 0

    lax.fori_loop(0, kcnt, _drain, 0)
    plsc.subcore_barrier()

    @pl.when(cid == 0)
    def _():
        pltpu.sync_copy(
            acc.at[pl.ds(sid * ROWS_PER_TILE, ROWS_PER_TILE)],
            out0_hbm.at[pl.ds(sid * ROWS_PER_TILE, ROWS_PER_TILE)])

    @pl.when(cid == 1)
    def _():
        pltpu.sync_copy(
            acc.at[pl.ds(sid * ROWS_PER_TILE, ROWS_PER_TILE)],
            out1_hbm.at[pl.ds(sid * ROWS_PER_TILE, ROWS_PER_TILE)])


def _make_agg(D, NB, CHK, KF, KS, ZR):

    @functools.partial(
        pl.kernel,
        mesh=_MESH,
        compiler_params=pltpu.CompilerParams(use_tc_tiling_on_sc=False),
        out_type=[jax.ShapeDtypeStruct((N_PAD, D), jnp.float32),
                  jax.ShapeDtypeStruct((N_PAD, D), jnp.float32)],
        scratch_types=[
            pltpu.VMEM((NB, CHK), jnp.int32),
            pltpu.VMEM((KF, CHK), jnp.int32),
            pltpu.VMEM((NB, CHK), jnp.float32),
            pltpu.VMEM((NB, CHK, D), jnp.float32),
            pltpu.VMEM((ZR, D), jnp.float32),
            pltpu.VMEM_SHARED((N_PAD, D), jnp.float32),
        ] + [pltpu.SemaphoreType.DMA] * (3 * NB),
    )
    def _agg(h_hbm, src_hbm, dst_hbm, ew_hbm, out0_hbm, out1_hbm,
             sidx, didx, ewr, msg, zbuf, acc, *sems):
        sem_i = sems[0:NB]
        sem_g = sems[NB:2 * NB]
        sem_s = sems[2 * NB:3 * NB]
        cid = lax.axis_index("c")
        sid = lax.axis_index("s")
        kbase, kcnt = _split(cid, sid, KF, KS)

        for i in range(ZR):
            for j in range(D // L):
                zbuf[i, pl.ds(j * L, L)] = jnp.zeros((L,), jnp.float32)

        def _zacc(i, _):
            pltpu.sync_copy(
                zbuf, acc.at[pl.ds(sid * ROWS_PER_TILE + i * ZR, ZR)])
            return 0

        lax.fori_loop(0, ROWS_PER_TILE // ZR, _zacc, 0)
        pltpu.sync_copy(dst_hbm.at[pl.ds(kbase, KF)], didx)

        def _fire_idx(k, b):
            pltpu.async_copy(src_hbm.at[kbase + k], sidx.at[b], sem_i[b])
            pltpu.async_copy(ew_hbm.at[kbase + k], ewr.at[b], sem_i[b])

        def _wait_idx(k, b):
            pltpu.make_async_copy(src_hbm.at[kbase + k], sidx.at[b],
                                  sem_i[b]).wait()
            pltpu.make_async_copy(ew_hbm.at[kbase + k], ewr.at[b],
                                  sem_i[b]).wait()

        def _fire_gather(b):
            pltpu.async_copy(h_hbm.at[sidx.at[b]], msg.at[b], sem_g[b])

        def _wait_gather(b):
            pltpu.make_async_copy(h_hbm.at[sidx.at[b]], msg.at[b],
                                  sem_g[b]).wait()

        def _fire_scatter(k, b):
            pltpu.async_copy(msg.at[b], acc.at[didx.at[k]], sem_s[b],
                             add=True)

        def _wait_scatter(k, b):
            pltpu.make_async_copy(msg.at[b], acc.at[didx.at[k]],
                                  sem_s[b]).wait()

        plsc.subcore_barrier()

        for b in range(NB):
            _fire_idx(b, b)
        _wait_idx(0, 0)
        _fire_gather(0)

        def _round(q, _):
            for i in range(NB):
                k = q * NB + i
                b = i
                b1 = (i + 1) % NB
                _wait_gather(b)

                @pl.when(k < kcnt - 1)
                def _(k=k, b1=b1):
                    _wait_idx(k + 1, b1)

                    @pl.when(k + 1 - NB >= 0)
                    def _():
                        _wait_scatter(k + 1 - NB, b1)

                    _fire_gather(b1)

                def _group(g, _, b=b):
                    w = ewr[b, pl.ds(g * L, L)]
                    for r in range(L):
                        s = w[r]
                        e = g * L + r
                        for j in range(D // L):
                            sl = pl.ds(j * L, L)
                            msg[b, e, sl] = msg[b, e, sl] * s
                    return 0

                lax.fori_loop(0, CHK // L, _group, 0)
                _fire_scatter(k, b)

                @pl.when(k + NB < kcnt)
                def _(k=k, b=b):
                    _fire_idx(k + NB, b)

            return 0

        lax.fori_loop(0, kcnt // NB, _round, 0)
        for i in range(NB):
            _wait_scatter(kcnt - NB + i, i)
        plsc.subcore_barrier()

        @pl.when(cid == 0)
        def _():
            pltpu.sync_copy(
                acc.at[pl.ds(sid * ROWS_PER_TILE, ROWS_PER_TILE)],
                out0_hbm.at[pl.ds(sid * ROWS_PER_TILE, ROWS_PER_TILE)])

        @pl.when(cid == 1)
        def _():
            pltpu.sync_copy(
                acc.at[pl.ds(sid * ROWS_PER_TILE, ROWS_PER_TILE)],
                out1_hbm.at[pl.ds(sid * ROWS_PER_TILE, ROWS_PER_TILE)])

    return _agg


_agg128 = _make_agg(D_HID, 4, 64, KF64, KS64, 4)
_agg64 = _make_agg(N_CLASSES, 8, CH, KF128, KS128, 8)


_BLK = 512
_NBLK = N_PAD // _BLK

_P = lax.Precision.HIGHEST


def _pre1_body(deg0_ref, deg1_ref, x_ref, w1_ref, dis_ref, h1p_ref):
    deg = deg0_ref[...] + deg1_ref[...]
    dis = jnp.where(deg > 0, lax.rsqrt(jnp.maximum(deg, 1e-12)), 0.0)
    dis_ref[...] = dis
    h = jnp.dot(x_ref[...], w1_ref[...], preferred_element_type=jnp.float32,
                precision=_P)
    h1p_ref[...] = h * dis[:, None]


def _pre1(deg0, deg1, x, w1):
    return pl.pallas_call(
        _pre1_body,
        grid=(_NBLK,),
        in_specs=[
            pl.BlockSpec((_BLK,), lambda i: (i,)),
            pl.BlockSpec((_BLK,), lambda i: (i,)),
            pl.BlockSpec((_BLK, D_IN), lambda i: (i, 0)),
            pl.BlockSpec((D_IN, D_HID), lambda i: (0, 0)),
        ],
        out_specs=[
            pl.BlockSpec((_BLK,), lambda i: (i,)),
            pl.BlockSpec((_BLK, D_HID), lambda i: (i, 0)),
        ],
        out_shape=[
            jax.ShapeDtypeStruct((N_PAD,), jnp.float32),
            jax.ShapeDtypeStruct((N_PAD, D_HID), jnp.float32),
        ],
    )(deg0, deg1, x, w1)


def _mid_body(a0_ref, a1_ref, dis_ref, b1_ref, w2_ref, out1_ref, h2p_ref):
    dis = dis_ref[...]
    t = (a0_ref[...] + a1_ref[...]) * dis[:, None] + b1_ref[...]
    out1 = jnp.maximum(t, 0.0)
    out1_ref[...] = out1
    h2 = jnp.dot(out1, w2_ref[...], preferred_element_type=jnp.float32,
                 precision=_P)
    h2p_ref[...] = h2 * dis[:, None]


def _mid(a0, a1, dis, b1, w2):
    return pl.pallas_call(
        _mid_body,
        grid=(_NBLK,),
        in_specs=[
            pl.BlockSpec((_BLK, D_HID), lambda i: (i, 0)),
            pl.BlockSpec((_BLK, D_HID), lambda i: (i, 0)),
            pl.BlockSpec((_BLK,), lambda i: (i,)),
            pl.BlockSpec((1, D_HID), lambda i: (0, 0)),
            pl.BlockSpec((D_HID, N_CLASSES), lambda i: (0, 0)),
        ],
        out_specs=[
            pl.BlockSpec((_BLK, D_HID), lambda i: (i, 0)),
            pl.BlockSpec((_BLK, N_CLASSES), lambda i: (i, 0)),
        ],
        out_shape=[
            jax.ShapeDtypeStruct((N_NODES, D_HID), jnp.float32),
            jax.ShapeDtypeStruct((N_PAD, N_CLASSES), jnp.float32),
        ],
    )(a0, a1, dis, b1, w2)


def _post_body(a0_ref, a1_ref, dis_ref, b2_ref, out2_ref):
    dis = dis_ref[...]
    out2_ref[...] = (a0_ref[...] + a1_ref[...]) * dis[:, None] + b2_ref[...]


def _post(a0, a1, dis, b2):
    return pl.pallas_call(
        _post_body,
        grid=(_NBLK,),
        in_specs=[
            pl.BlockSpec((_BLK, N_CLASSES), lambda i: (i, 0)),
            pl.BlockSpec((_BLK, N_CLASSES), lambda i: (i, 0)),
            pl.BlockSpec((_BLK,), lambda i: (i,)),
            pl.BlockSpec((1, N_CLASSES), lambda i: (0, 0)),
        ],
        out_specs=pl.BlockSpec((_BLK, N_CLASSES), lambda i: (i, 0)),
        out_shape=jax.ShapeDtypeStruct((N_NODES, N_CLASSES), jnp.float32),
    )(a0, a1, dis, b2)


def kernel(x, edge_index, edge_weight, W1, b1, W2, b2):
    src = edge_index[0].astype(jnp.int32)
    dst = edge_index[1].astype(jnp.int32)
    ew = edge_weight.astype(jnp.float32)
    epad = E_PAD - N_EDGES
    tpad = E_TOT - E_PAD
    src = jnp.concatenate([src, jnp.zeros((epad + tpad,), jnp.int32)])
    dump = N_NODES + jnp.mod(jnp.arange(epad, dtype=jnp.int32),
                             N_PAD - N_NODES)
    dst = jnp.concatenate([dst, dump, jnp.zeros((tpad,), jnp.int32)])
    ew = jnp.concatenate([ew, jnp.zeros((epad + tpad,), jnp.float32)])
    src128 = src.reshape(-1, CH)
    dst128 = dst.reshape(-1, CH)
    ew128 = ew.reshape(-1, CH)
    src64 = src.reshape(-1, CH // 2)
    dst64 = dst.reshape(-1, CH // 2)
    ew64 = ew.reshape(-1, CH // 2)
    x_pad = jnp.pad(x, ((0, N_PAD - N_NODES), (0, 0)))

    deg0, deg1 = _deg_kernel(dst128, ew128)
    dis, h1p = _pre1(deg0, deg1, x_pad, W1)
    a0, a1 = _agg128(h1p, src64, dst64, ew64)
    out1, h2p = _mid(a0, a1, dis, b1.reshape(1, D_HID), W2)
    q0, q1 = _agg64(h2p, src128, dst128, ew128)
    out2 = _post(q0, q1, dis, b2.reshape(1, N_CLASSES))
    return (x, out1, out2)
